# Initial kernel scaffold; baseline (speedup 1.0000x reference)
import functools

import jax
import jax.numpy as jnp
from jax import lax
from jax.experimental import pallas as pl
from jax.experimental.pallas import tpu as pltpu
from jax.experimental.pallas import tpu_sc as plsc

NUM_EMBEDDINGS = 1000000
EMBEDDING_DIM = 64
BATCH = 16384
FIELDS = 100

_B = BATCH * FIELDS
_NC = 2
_NS = 16
_NW = _NC * _NS
_B_PER_W = _B // _NW
_CHUNK = 128
_N_CHUNKS = _B_PER_W // _CHUNK


def _emb_body(x_hbm, w_hbm, out_hbm, idx_v, rows_v, gsem):
    wid = lax.axis_index("s") * _NC + lax.axis_index("c")
    base = wid * _B_PER_W

    @pl.loop(0, _N_CHUNKS)
    def _chunk(g):
        off = base + g * _CHUNK
        pltpu.sync_copy(x_hbm.at[pl.ds(off, _CHUNK)], idx_v)
        pltpu.async_copy(w_hbm.at[idx_v], rows_v, gsem).wait()
        pltpu.sync_copy(rows_v, out_hbm.at[pl.ds(off, _CHUNK)])


_emb = functools.partial(
    pl.kernel,
    out_type=jax.ShapeDtypeStruct((_B, EMBEDDING_DIM), jnp.float32),
    mesh=plsc.VectorSubcoreMesh(core_axis_name="c", subcore_axis_name="s"),
    scratch_types=[
        pltpu.VMEM((_CHUNK,), jnp.int32),
        pltpu.VMEM((_CHUNK, EMBEDDING_DIM), jnp.float32),
        pltpu.SemaphoreType.DMA,
    ],
)(_emb_body)


@jax.jit
def kernel(x, weight):
    out = _emb(x.reshape(_B), weight)
    return out.reshape(BATCH, FIELDS, EMBEDDING_DIM)


# speedup vs baseline: 4.5024x; 4.5024x over previous
import functools

import jax
import jax.numpy as jnp
from jax import lax
from jax.experimental import pallas as pl
from jax.experimental.pallas import tpu as pltpu
from jax.experimental.pallas import tpu_sc as plsc

NUM_EMBEDDINGS = 1000000
EMBEDDING_DIM = 64
BATCH = 16384
FIELDS = 100

_B = BATCH * FIELDS
_NC = 2
_NS = 16
_NW = _NC * _NS
_B_PER_W = _B // _NW
_CHUNK = 128
_N_CHUNKS = _B_PER_W // _CHUNK


def _emb_body(x_hbm, w_hbm, out_hbm, idx_v, rows_v, gsem):
    wid = lax.axis_index("s") * _NC + lax.axis_index("c")
    base = wid * _B_PER_W

    @pl.loop(0, _N_CHUNKS)
    def _chunk(g):
        off = base + g * _CHUNK
        pltpu.sync_copy(x_hbm.at[pl.ds(off, _CHUNK)], idx_v)
        pltpu.async_copy(w_hbm.at[idx_v], rows_v, gsem).wait()
        pltpu.sync_copy(rows_v, out_hbm.at[pl.ds(off, _CHUNK)])


_emb = functools.partial(
    pl.kernel,
    out_type=jax.ShapeDtypeStruct((_B, EMBEDDING_DIM), jnp.float32),
    mesh=plsc.VectorSubcoreMesh(core_axis_name="c", subcore_axis_name="s"),
    scratch_types=[
        pltpu.VMEM((_CHUNK,), jnp.int32),
        pltpu.VMEM((_CHUNK, EMBEDDING_DIM), jnp.float32),
        pltpu.SemaphoreType.DMA,
    ],
    compiler_params=pltpu.CompilerParams(use_tc_tiling_on_sc=False),
)(_emb_body)


@jax.jit
def kernel(x, weight):
    out = _emb(x.reshape(_B), weight)
    return out.reshape(BATCH, FIELDS, EMBEDDING_DIM)


# speedup vs baseline: 5.6556x; 1.2561x over previous
import functools

import jax
import jax.numpy as jnp
from jax import lax
from jax.experimental import pallas as pl
from jax.experimental.pallas import tpu as pltpu
from jax.experimental.pallas import tpu_sc as plsc

NUM_EMBEDDINGS = 1000000
EMBEDDING_DIM = 64
BATCH = 16384
FIELDS = 100

_B = BATCH * FIELDS
_NC = 2
_NS = 16
_NW = _NC * _NS
_B_PER_W = _B // _NW
_CHUNK = 128
_N_CHUNKS = _B_PER_W // _CHUNK
_NBUF = 4


def _emb_body(x_hbm, w_hbm, out_hbm, idx_v, rows_v, gsems):
    wid = lax.axis_index("s") * _NC + lax.axis_index("c")
    base = wid * _B_PER_W

    pltpu.sync_copy(x_hbm.at[wid], idx_v)

    def fire(g, b):
        return pltpu.async_copy(w_hbm.at[idx_v.at[g]], rows_v.at[b], gsems[b])

    def drain(g, b):
        off = base + g * _CHUNK
        pltpu.make_async_copy(w_hbm.at[idx_v.at[g]], rows_v.at[b],
                              gsems[b]).wait()
        pltpu.sync_copy(rows_v.at[b], out_hbm.at[pl.ds(off, _CHUNK)])

    for b in range(_NBUF):
        fire(b, b)

    @pl.loop(0, _N_CHUNKS // _NBUF - 1)
    def _grp(gg):
        go = gg * _NBUF
        for b in range(_NBUF):
            g = go + b
            drain(g, b)
            fire(g + _NBUF, b)

    for b in range(_NBUF):
        drain(_N_CHUNKS - _NBUF + b, b)


_emb = functools.partial(
    pl.kernel,
    out_type=jax.ShapeDtypeStruct((_B, EMBEDDING_DIM), jnp.float32),
    mesh=plsc.VectorSubcoreMesh(core_axis_name="c", subcore_axis_name="s"),
    scratch_types=[
        pltpu.VMEM((_N_CHUNKS, _CHUNK), jnp.int32),
        pltpu.VMEM((_NBUF, _CHUNK, EMBEDDING_DIM), jnp.float32),
        [pltpu.SemaphoreType.DMA] * _NBUF,
    ],
    compiler_params=pltpu.CompilerParams(use_tc_tiling_on_sc=False),
)(_emb_body)


@jax.jit
def kernel(x, weight):
    out = _emb(x.reshape(_NW, _N_CHUNKS, _CHUNK), weight)
    return out.reshape(BATCH, FIELDS, EMBEDDING_DIM)
